# baseline (speedup 1.0000x reference)
import jax
import jax.numpy as jnp
from jax.experimental import pallas as pl
from jax.experimental.pallas import tpu as pltpu

_N = 100000
_CHUNK = 2000
_NBUF = 8
_NCHUNKS = _N // _CHUNK


def _mm_bias_kernel(x_hbm, w_ref, b_ref, o_hbm, xbuf, obuf, isem, osem):
    w = w_ref[...]
    b = b_ref[...]

    def in_cp(slot, idx):
        return pltpu.make_async_copy(
            x_hbm.at[pl.ds(idx * _CHUNK, _CHUNK), :],
            xbuf.at[slot],
            isem.at[slot],
        )

    def out_cp(slot, idx):
        return pltpu.make_async_copy(
            obuf.at[slot],
            o_hbm.at[pl.ds(idx * _CHUNK, _CHUNK), :],
            osem.at[slot],
        )

    for s in range(_NBUF):
        in_cp(s, s).start()

    def body(i, carry):
        slot = jax.lax.rem(i, _NBUF)
        in_cp(slot, i).wait()

        @pl.when(i >= _NBUF)
        def _():
            out_cp(slot, i - _NBUF).wait()

        obuf[slot] = (
            jnp.dot(xbuf[slot], w, preferred_element_type=jnp.float32) + b
        )
        out_cp(slot, i).start()

        @pl.when(i + _NBUF < _NCHUNKS)
        def _():
            in_cp(slot, i + _NBUF).start()

        return carry

    jax.lax.fori_loop(0, _NCHUNKS, body, 0)

    for idx in range(_NCHUNKS - _NBUF, _NCHUNKS):
        out_cp(idx % _NBUF, idx).wait()


def kernel(input, kernel, bias):
    n, in_ch = input.shape
    out_ch = kernel.shape[1]
    return pl.pallas_call(
        _mm_bias_kernel,
        in_specs=[
            pl.BlockSpec(memory_space=pltpu.HBM),
            pl.BlockSpec(memory_space=pltpu.VMEM),
            pl.BlockSpec(memory_space=pltpu.VMEM),
        ],
        out_specs=pl.BlockSpec(memory_space=pltpu.HBM),
        out_shape=jax.ShapeDtypeStruct((n, out_ch), jnp.float32),
        scratch_shapes=[
            pltpu.VMEM((_NBUF, _CHUNK, in_ch), jnp.float32),
            pltpu.VMEM((_NBUF, _CHUNK, out_ch), jnp.float32),
            pltpu.SemaphoreType.DMA((_NBUF,)),
            pltpu.SemaphoreType.DMA((_NBUF,)),
        ],
    )(input, kernel, bias)


# speedup vs baseline: 1.0098x; 1.0098x over previous
import jax
import jax.numpy as jnp
from jax.experimental import pallas as pl
from jax.experimental.pallas import tpu as pltpu

_N = 100000
_CHUNK = 10000
_NBUF = 3
_NCHUNKS = _N // _CHUNK


def _mm_bias_kernel(x_hbm, w_ref, b_ref, o_hbm, xbuf, obuf, isem, osem):
    w = w_ref[...]
    b = b_ref[...]

    def in_cp(slot, idx):
        return pltpu.make_async_copy(
            x_hbm.at[pl.ds(idx * _CHUNK, _CHUNK), :],
            xbuf.at[slot],
            isem.at[slot],
        )

    def out_cp(slot, idx):
        return pltpu.make_async_copy(
            obuf.at[slot],
            o_hbm.at[pl.ds(idx * _CHUNK, _CHUNK), :],
            osem.at[slot],
        )

    for s in range(_NBUF):
        in_cp(s, s).start()

    def body(i, carry):
        slot = jax.lax.rem(i, _NBUF)
        in_cp(slot, i).wait()

        @pl.when(i >= _NBUF)
        def _():
            out_cp(slot, i - _NBUF).wait()

        obuf[slot] = (
            jnp.dot(xbuf[slot], w, preferred_element_type=jnp.float32) + b
        )
        out_cp(slot, i).start()

        @pl.when(i + _NBUF < _NCHUNKS)
        def _():
            in_cp(slot, i + _NBUF).start()

        return carry

    jax.lax.fori_loop(0, _NCHUNKS, body, 0)

    for idx in range(_NCHUNKS - _NBUF, _NCHUNKS):
        out_cp(idx % _NBUF, idx).wait()


def kernel(input, kernel, bias):
    n, in_ch = input.shape
    out_ch = kernel.shape[1]
    return pl.pallas_call(
        _mm_bias_kernel,
        in_specs=[
            pl.BlockSpec(memory_space=pltpu.HBM),
            pl.BlockSpec(memory_space=pltpu.VMEM),
            pl.BlockSpec(memory_space=pltpu.VMEM),
        ],
        out_specs=pl.BlockSpec(memory_space=pltpu.HBM),
        out_shape=jax.ShapeDtypeStruct((n, out_ch), jnp.float32),
        scratch_shapes=[
            pltpu.VMEM((_NBUF, _CHUNK, in_ch), jnp.float32),
            pltpu.VMEM((_NBUF, _CHUNK, out_ch), jnp.float32),
            pltpu.SemaphoreType.DMA((_NBUF,)),
            pltpu.SemaphoreType.DMA((_NBUF,)),
        ],
    )(input, kernel, bias)


# speedup vs baseline: 1.0327x; 1.0226x over previous
import jax
import jax.numpy as jnp
from jax.experimental import pallas as pl
from jax.experimental.pallas import tpu as pltpu

_BLOCK = 16000


def _mm_bias_kernel(x_ref, w_ref, b_ref, o_ref):
    o_ref[...] = (
        jnp.dot(x_ref[...], w_ref[...], preferred_element_type=jnp.float32)
        + b_ref[...]
    )


def kernel(input, kernel, bias):
    n, in_ch = input.shape
    out_ch = kernel.shape[1]
    grid = pl.cdiv(n, _BLOCK)
    return pl.pallas_call(
        _mm_bias_kernel,
        grid=(grid,),
        in_specs=[
            pl.BlockSpec((_BLOCK, in_ch), lambda i: (i, 0)),
            pl.BlockSpec((in_ch, out_ch), lambda i: (0, 0)),
            pl.BlockSpec((1, out_ch), lambda i: (0, 0)),
        ],
        out_specs=pl.BlockSpec((_BLOCK, out_ch), lambda i: (i, 0)),
        out_shape=jax.ShapeDtypeStruct((n, out_ch), jnp.float32),
        compiler_params=pltpu.CompilerParams(
            dimension_semantics=("parallel",),
        ),
    )(input, kernel, bias)
